# baseline (speedup 1.0000x reference)
import jax
import jax.numpy as jnp
from jax import lax
from jax.experimental import pallas as pl
from jax.experimental.pallas import tpu as pltpu
from jax.experimental.pallas import tpu_sc as plsc

NUM_REL = 6
NUM_HEADS = 32
NUM_POS = 64 * 64
LANES = 16


def _sc_relation_bias(w, idx_flat):
    mesh = plsc.VectorSubcoreMesh(core_axis_name="c", subcore_axis_name="s")

    def body(w_hbm, idx_hbm, out_hbm, w_v, idx_v, out_v):
        h = lax.axis_index("s") * 2 + lax.axis_index("c")
        pltpu.sync_copy(w_hbm, w_v)
        pltpu.sync_copy(idx_hbm, idx_v)
        hvec = jnp.full((LANES,), h, dtype=jnp.int32)
        for t in range(NUM_POS // LANES):
            sl = pl.ds(t * LANES, LANES)
            flat = idx_v[sl] * NUM_HEADS + hvec
            out_v[sl] = plsc.load_gather(w_v, [flat])
        pltpu.sync_copy(out_v, out_hbm.at[h])

    return pl.kernel(
        body,
        mesh=mesh,
        compiler_params=pltpu.CompilerParams(needs_layout_passes=False),
        out_type=jax.ShapeDtypeStruct((NUM_HEADS, NUM_POS), jnp.float32),
        scratch_types=[
            pltpu.VMEM((NUM_REL * NUM_HEADS,), jnp.float32),
            pltpu.VMEM((NUM_POS,), jnp.int32),
            pltpu.VMEM((NUM_POS,), jnp.float32),
        ],
    )(w, idx_flat)


def kernel(embedding_weight, relation_index):
    w = embedding_weight.astype(jnp.float32).reshape(NUM_REL * NUM_HEADS)
    idx_flat = relation_index.reshape(NUM_POS).astype(jnp.int32)
    out = _sc_relation_bias(w, idx_flat)
    return out.reshape(NUM_HEADS, 64, 64)


# speedup vs baseline: 5.2619x; 5.2619x over previous
import jax
import jax.numpy as jnp
from jax.experimental import pallas as pl
from jax.experimental.pallas import tpu as pltpu

NUM_REL = 6
NUM_HEADS = 32


def _tc_body(w_ref, idx_ref, out_ref):
    idx = idx_ref[...]
    masks = [idx == r for r in range(1, NUM_REL)]
    for h in range(NUM_HEADS):
        acc = jnp.full((32, 128), w_ref[0, h], dtype=jnp.float32)
        for r in range(1, NUM_REL):
            acc = jnp.where(masks[r - 1], w_ref[r, h], acc)
        out_ref[h] = acc


def kernel(embedding_weight, relation_index):
    w = embedding_weight.astype(jnp.float32)
    idx = relation_index.astype(jnp.int32).reshape(32, 128)
    out = pl.pallas_call(
        _tc_body,
        out_shape=jax.ShapeDtypeStruct((NUM_HEADS, 32, 128), jnp.float32),
        in_specs=[
            pl.BlockSpec(memory_space=pltpu.SMEM),
            pl.BlockSpec(memory_space=pltpu.VMEM),
        ],
        out_specs=pl.BlockSpec(memory_space=pltpu.VMEM),
    )(w, idx)
    return out.reshape(NUM_HEADS, 64, 64)
